# baseline (speedup 1.0000x reference)
import jax
import jax.numpy as jnp
from jax import lax
from jax.experimental import pallas as pl
from jax.experimental.pallas import tpu as pltpu
from jax.experimental.pallas import tpu_sc as plsc

B = 16384
D = 100
DA = 128
NC = 2
NS = 16
NW = NC * NS
CHUNK = B // NW
QC = 128
NQ = CHUNK // QC
QGROUPS = QC // 16
NFULL = 96 // 16
WIN = 96

_GDN = lax.GatherDimensionNumbers(
    offset_dims=(), collapsed_slice_dims=(0,), start_index_map=(0,))


def _xor_shuffle(x, perm):
    return lax.gather(x, perm, _GDN, (1,),
                      mode=lax.GatherScatterMode.PROMISE_IN_BOUNDS)


def _sc_body(uid_hbm, iid_hbm, uaux_hbm, iaux_hbm, out_hbm,
             uidx_v, iidx_v, urows0_v, urows1_v, irows0_v, irows1_v, out_v, sem):
    wid = lax.axis_index("s") * NC + lax.axis_index("c")
    base = wid * CHUNK

    for j in range(NQ):
        pltpu.sync_copy(uid_hbm.at[pl.ds(base + j * QC, QC)], uidx_v.at[j])
        pltpu.sync_copy(iid_hbm.at[pl.ds(base + j * QC, QC)], iidx_v.at[j])

    def fire(q):
        ur = urows0_v if q % 2 == 0 else urows1_v
        ir = irows0_v if q % 2 == 0 else irows1_v
        return (pltpu.async_copy(uaux_hbm.at[uidx_v.at[q]], ur, sem),
                pltpu.async_copy(iaux_hbm.at[iidx_v.at[q]], ir, sem))

    for q in range(NQ):
        for c in fire(q):
            c.wait()
        ur = urows0_v if q % 2 == 0 else urows1_v
        ir = irows0_v if q % 2 == 0 else irows1_v

        def group_body(g, _, ur=ur, ir=ir, q=q):
            lane = lax.iota(jnp.int32, 16)
            dots = jnp.zeros((16,), jnp.float32)
            for e in range(16):
                b = g * 16 + e
                acc = ur[b, pl.ds(0, 16)] * ir[b, pl.ds(0, 16)]
                for c in range(1, NFULL):
                    acc = acc + (ur[b, pl.ds(c * 16, 16)]
                                 * ir[b, pl.ds(c * 16, 16)])
                acc = acc + ur[b, pl.ds(WIN, 16)] * ir[b, pl.ds(WIN, 16)]
                for s in (8, 4, 2, 1):
                    acc = acc + _xor_shuffle(acc, (lane ^ s).reshape(16, 1))
                dots = jnp.where(lane == e, acc, dots)
            out_v[pl.ds(q * QC + g * 16, 16)] = dots
            return 0

        lax.fori_loop(0, QGROUPS, group_body, 0)

    pltpu.sync_copy(out_v, out_hbm.at[0, pl.ds(base, CHUNK)])


@jax.jit
def _net_sc(uid, iid, user_emb, user_basis, item_emb, item_basis):
    n_u = user_emb.shape[0]
    n_i = item_emb.shape[0]
    uaux = jnp.concatenate(
        [user_emb, user_basis, jnp.ones((n_u, 1), jnp.float32),
         jnp.zeros((n_u, DA - D - 2), jnp.float32)], axis=1)
    iaux = jnp.concatenate(
        [item_emb, jnp.ones((n_i, 1), jnp.float32), item_basis,
         jnp.zeros((n_i, DA - D - 2), jnp.float32)], axis=1)

    run = pl.kernel(
        _sc_body,
        out_type=jax.ShapeDtypeStruct((1, B), jnp.float32),
        mesh=plsc.VectorSubcoreMesh(core_axis_name="c", subcore_axis_name="s"),
        compiler_params=pltpu.CompilerParams(use_tc_tiling_on_sc=False),
        scratch_types=[
            pltpu.VMEM((NQ, QC), jnp.int32),
            pltpu.VMEM((NQ, QC), jnp.int32),
            pltpu.VMEM((QC, DA), jnp.float32),
            pltpu.VMEM((QC, DA), jnp.float32),
            pltpu.VMEM((QC, DA), jnp.float32),
            pltpu.VMEM((QC, DA), jnp.float32),
            pltpu.VMEM((CHUNK,), jnp.float32),
            pltpu.SemaphoreType.DMA,
        ],
    )
    return run(uid, iid, uaux, iaux)


def kernel(uid, iid, user_emb, user_basis, item_emb, item_basis):
    return _net_sc(uid, iid, user_emb, user_basis, item_emb, item_basis)


# speedup vs baseline: 1.9004x; 1.9004x over previous
import jax
import jax.numpy as jnp
from jax import lax
from jax.experimental import pallas as pl
from jax.experimental.pallas import tpu as pltpu
from jax.experimental.pallas import tpu_sc as plsc

B = 16384
D = 100
DA = 128
NC = 2
NS = 16
NW = NC * NS
CHUNK = B // NW
QC = 128
NQ = CHUNK // QC
QGROUPS = QC // 16
NFULL = 96 // 16
WIN = 96

_GDN = lax.GatherDimensionNumbers(
    offset_dims=(), collapsed_slice_dims=(0,), start_index_map=(0,))


def _xor_shuffle(x, perm):
    return lax.gather(x, perm, _GDN, (1,),
                      mode=lax.GatherScatterMode.PROMISE_IN_BOUNDS)


def _sc_body(uid_hbm, iid_hbm, uaux_hbm, iaux_hbm, out_hbm,
             uidx_v, iidx_v, urows0_v, urows1_v, irows0_v, irows1_v, out_v, sem):
    wid = lax.axis_index("s") * NC + lax.axis_index("c")
    base = wid * CHUNK

    for j in range(NQ):
        pltpu.sync_copy(uid_hbm.at[pl.ds(base + j * QC, QC)], uidx_v.at[j])
        pltpu.sync_copy(iid_hbm.at[pl.ds(base + j * QC, QC)], iidx_v.at[j])

    def fire(q):
        ur = urows0_v if q % 2 == 0 else urows1_v
        ir = irows0_v if q % 2 == 0 else irows1_v
        return (pltpu.async_copy(uaux_hbm.at[uidx_v.at[q]], ur, sem),
                pltpu.async_copy(iaux_hbm.at[iidx_v.at[q]], ir, sem))

    for q in range(NQ):
        for c in fire(q):
            c.wait()
        ur = urows0_v if q % 2 == 0 else urows1_v
        ir = irows0_v if q % 2 == 0 else irows1_v

        def group_body(g, _, ur=ur, ir=ir, q=q):
            lane = lax.iota(jnp.int32, 16)
            dots = jnp.zeros((16,), jnp.float32)
            for e in range(16):
                b = g * 16 + e
                acc = ur[b, pl.ds(0, 16)] * ir[b, pl.ds(0, 16)]
                for c in range(1, NFULL):
                    acc = acc + (ur[b, pl.ds(c * 16, 16)]
                                 * ir[b, pl.ds(c * 16, 16)])
                acc = acc + ur[b, pl.ds(WIN, 16)] * ir[b, pl.ds(WIN, 16)]
                for s in (8, 4, 2, 1):
                    acc = acc + _xor_shuffle(acc, (lane ^ s).reshape(16, 1))
                dots = jnp.where(lane == e, acc, dots)
            out_v[pl.ds(q * QC + g * 16, 16)] = dots
            return 0

        lax.fori_loop(0, QGROUPS, group_body, 0)

    pltpu.sync_copy(out_v, out_hbm.at[0, pl.ds(base, CHUNK)])


AUX_BR = 2048


def _aux_body(uemb_ref, ubas_ref, iemb_ref, ibas_ref, uaux_ref, iaux_ref):
    lanec = lax.broadcasted_iota(jnp.int32, (AUX_BR, DA - D), 1) + D
    um = jnp.where(lanec == D, ubas_ref[...], 0.0)
    um = jnp.where(lanec == D + 1, 1.0, um)
    im = jnp.where(lanec == D, 1.0, 0.0)
    im = jnp.where(lanec == D + 1, ibas_ref[...], im)
    uaux_ref[...] = jnp.concatenate([uemb_ref[...], um], axis=1)
    iaux_ref[...] = jnp.concatenate([iemb_ref[...], im], axis=1)


def _build_aux(user_emb, user_basis, item_emb, item_basis):
    n = user_emb.shape[0]
    grid = (n + AUX_BR - 1) // AUX_BR
    return pl.pallas_call(
        _aux_body,
        grid=(grid,),
        in_specs=[
            pl.BlockSpec((AUX_BR, D), lambda i: (i, 0)),
            pl.BlockSpec((AUX_BR, 1), lambda i: (i, 0)),
            pl.BlockSpec((AUX_BR, D), lambda i: (i, 0)),
            pl.BlockSpec((AUX_BR, 1), lambda i: (i, 0)),
        ],
        out_specs=[
            pl.BlockSpec((AUX_BR, DA), lambda i: (i, 0)),
            pl.BlockSpec((AUX_BR, DA), lambda i: (i, 0)),
        ],
        out_shape=[
            jax.ShapeDtypeStruct((n, DA), jnp.float32),
            jax.ShapeDtypeStruct((n, DA), jnp.float32),
        ],
    )(user_emb, user_basis, item_emb, item_basis)


@jax.jit
def _net_sc(uid, iid, user_emb, user_basis, item_emb, item_basis):
    uaux, iaux = _build_aux(user_emb, user_basis, item_emb, item_basis)

    run = pl.kernel(
        _sc_body,
        out_type=jax.ShapeDtypeStruct((1, B), jnp.float32),
        mesh=plsc.VectorSubcoreMesh(core_axis_name="c", subcore_axis_name="s"),
        compiler_params=pltpu.CompilerParams(use_tc_tiling_on_sc=False),
        scratch_types=[
            pltpu.VMEM((NQ, QC), jnp.int32),
            pltpu.VMEM((NQ, QC), jnp.int32),
            pltpu.VMEM((QC, DA), jnp.float32),
            pltpu.VMEM((QC, DA), jnp.float32),
            pltpu.VMEM((QC, DA), jnp.float32),
            pltpu.VMEM((QC, DA), jnp.float32),
            pltpu.VMEM((CHUNK,), jnp.float32),
            pltpu.SemaphoreType.DMA,
        ],
    )
    return run(uid, iid, uaux, iaux)


def kernel(uid, iid, user_emb, user_basis, item_emb, item_basis):
    return _net_sc(uid, iid, user_emb, user_basis, item_emb, item_basis)
